# baseline (speedup 1.0000x reference)
import jax
import jax.numpy as jnp
from jax.experimental import pallas as pl

_F32 = jnp.float32


def _make_body(B, N, P, D, DOUT):
    PD = P * D
    M = B * N * P

    def body(x1f_ref, x2f_ref, x1r_ref, x2r_ref,
             wx_ref, wn_ref, wr_ref,
             bx_ref, bn_ref, br_ref, g_ref, bt_ref,
             o1_ref, o2_ref):
        mu1_parts = []
        mu2_parts = []
        for b in range(B):
            A1 = x1f_ref[b]
            A2 = x2f_ref[b]
            num = jax.lax.dot_general(
                A1, A2, (((1,), (1,)), ((), ())), preferred_element_type=_F32)
            n1 = jnp.maximum(
                jnp.sqrt(jnp.sum(A1 * A1, axis=1, keepdims=True)), 1e-6)
            n2 = jnp.maximum(
                jnp.sqrt(jnp.sum(A2 * A2, axis=1, keepdims=True)), 1e-6)
            sim = num / (n1 * n2.T)
            m1 = jnp.max(sim, axis=1, keepdims=True)
            e1 = jnp.exp(sim - m1)
            att1 = e1 / jnp.sum(e1, axis=1, keepdims=True)
            m2 = jnp.max(sim, axis=0, keepdims=True)
            e2 = jnp.exp(sim - m2)
            att2t = e2 / jnp.sum(e2, axis=0, keepdims=True)
            ca1 = jnp.dot(att1, A2, preferred_element_type=_F32)
            ca2 = jax.lax.dot_general(
                att2t, A1, (((0,), (0,)), ((), ())), preferred_element_type=_F32)
            mu1_parts.append((A1 - ca1).reshape(N * P, D))
            mu2_parts.append((A2 - ca2).reshape(N * P, D))
        MU1 = jnp.concatenate(mu1_parts, axis=0)
        MU2 = jnp.concatenate(mu2_parts, axis=0)

        wx = wx_ref[:]
        wn = wn_ref[:]
        wr = wr_ref[:]
        bx = bx_ref[:]
        bn = bn_ref[:]
        br = br_ref[:]
        inv = 1.0 / (N - 1)
        for X, MU, oref in ((x1r_ref[:], MU1, o1_ref),
                            (x2r_ref[:], MU2, o2_ref)):
            oref[:, 0:DOUT] = jnp.dot(X, wx, preferred_element_type=_F32) + bx
            Z = jnp.dot(X, wn, preferred_element_type=_F32)
            Z4 = Z.reshape(B, N, P, DOUT)
            s = jnp.sum(Z4, axis=1, keepdims=True)
            oref[:, DOUT:2 * DOUT] = ((s - Z4) * inv).reshape(M, DOUT) + bn
            oref[:, 2 * DOUT:3 * DOUT] = (
                jnp.dot(MU, wr, preferred_element_type=_F32) + br)

        g = g_ref[:]
        bt = bt_ref[:]
        for oref in (o1_ref, o2_ref):
            h = oref[:]
            nrm = jnp.sqrt(jnp.sum(h * h, axis=1, keepdims=True))
            h = h / jnp.maximum(nrm, 1e-12)
            h = jnp.maximum(h, 0.0)
            mean = jnp.mean(h, axis=0, keepdims=True)
            var = jnp.mean((h - mean) ** 2, axis=0, keepdims=True)
            oref[:] = g * (h - mean) * jax.lax.rsqrt(var + 1e-5) + bt

    return body


def kernel(x1, x2, adj1, adj2, Wx_w, Wx_b, Wn_w, Wn_b, Wr_w, Wr_b, gamma, beta):
    B, N, P, D = x1.shape
    DOUT = Wx_w.shape[0]
    C = 3 * DOUT
    M = B * N * P

    x1f = x1.reshape(B, N, P * D)
    x2f = x2.reshape(B, N, P * D)
    x1r = x1.reshape(M, D)
    x2r = x2.reshape(M, D)

    out1, out2 = pl.pallas_call(
        _make_body(B, N, P, D, DOUT),
        out_shape=(
            jax.ShapeDtypeStruct((M, C), jnp.float32),
            jax.ShapeDtypeStruct((M, C), jnp.float32),
        ),
    )(x1f, x2f, x1r, x2r,
      Wx_w.T, Wn_w.T, Wr_w.T,
      Wx_b.reshape(1, DOUT), Wn_b.reshape(1, DOUT), Wr_b.reshape(1, DOUT),
      gamma.reshape(1, C), beta.reshape(1, C))

    return (out1.reshape(B, N, P, C), out2.reshape(B, N, P, C))


# speedup vs baseline: 1.1827x; 1.1827x over previous
import jax
import jax.numpy as jnp
from jax.experimental import pallas as pl

_F32 = jnp.float32


def _make_body(B, N, P, D, DOUT):
    C = 3 * DOUT
    BN_CNT = B * N * P

    def body(x1_ref, x2_ref, wx_ref, wn_ref, wr_ref,
             bx_ref, bn_ref, br_ref, g_ref, bt_ref,
             o1_ref, o2_ref):
        mu1_parts = []
        mu2_parts = []
        for b in range(B):
            A1 = x1_ref[b * N:(b + 1) * N, :]
            A2 = x2_ref[b * N:(b + 1) * N, :]
            num = jax.lax.dot_general(
                A1, A2, (((1,), (1,)), ((), ())), preferred_element_type=_F32)
            n1 = jnp.maximum(
                jnp.sqrt(jnp.sum(A1 * A1, axis=1, keepdims=True)), 1e-6)
            n2 = jnp.maximum(
                jnp.sqrt(jnp.sum(A2 * A2, axis=1, keepdims=True)), 1e-6)
            sim = num / (n1 * n2.T)
            m1 = jnp.max(sim, axis=1, keepdims=True)
            e1 = jnp.exp(sim - m1)
            att1 = e1 / jnp.sum(e1, axis=1, keepdims=True)
            m2 = jnp.max(sim, axis=0, keepdims=True)
            e2 = jnp.exp(sim - m2)
            att2t = e2 / jnp.sum(e2, axis=0, keepdims=True)
            ca1 = jnp.dot(att1, A2, preferred_element_type=_F32)
            ca2 = jax.lax.dot_general(
                att2t, A1, (((0,), (0,)), ((), ())), preferred_element_type=_F32)
            mu1_parts.append(A1 - ca1)
            mu2_parts.append(A2 - ca2)
        MU1 = jnp.concatenate(mu1_parts, axis=0)
        MU2 = jnp.concatenate(mu2_parts, axis=0)

        wx = wx_ref[:]
        wn = wn_ref[:]
        wr = wr_ref[:]
        bx = bx_ref[:]
        bn = bn_ref[:]
        br = br_ref[:]
        inv = 1.0 / (N - 1)
        for X, MU, oref in ((x1_ref[:], MU1, o1_ref),
                            (x2_ref[:], MU2, o2_ref)):
            for p in range(P):
                Xp = X[:, p * D:(p + 1) * D]
                oref[:, p * C:p * C + DOUT] = (
                    jnp.dot(Xp, wx, preferred_element_type=_F32) + bx)
                Z = jnp.dot(Xp, wn, preferred_element_type=_F32)
                Z3 = Z.reshape(B, N, DOUT)
                s = jnp.sum(Z3, axis=1, keepdims=True)
                oref[:, p * C + DOUT:p * C + 2 * DOUT] = (
                    ((s - Z3) * inv).reshape(B * N, DOUT) + bn)
                oref[:, p * C + 2 * DOUT:(p + 1) * C] = (
                    jnp.dot(MU[:, p * D:(p + 1) * D], wr,
                            preferred_element_type=_F32) + br)

        g = g_ref[:]
        bt = bt_ref[:]
        for oref in (o1_ref, o2_ref):
            h = oref[:]
            hs = [h[:, p * C:(p + 1) * C] for p in range(P)]
            hn = []
            for p in range(P):
                nrm = jnp.sqrt(jnp.sum(hs[p] * hs[p], axis=1, keepdims=True))
                hn.append(jnp.maximum(hs[p] / jnp.maximum(nrm, 1e-12), 0.0))
            tot = jnp.zeros((1, C), _F32)
            for p in range(P):
                tot = tot + jnp.sum(hn[p], axis=0, keepdims=True)
            mean = tot * (1.0 / BN_CNT)
            totv = jnp.zeros((1, C), _F32)
            for p in range(P):
                dlt = hn[p] - mean
                hn[p] = dlt
                totv = totv + jnp.sum(dlt * dlt, axis=0, keepdims=True)
            scale = g * jax.lax.rsqrt(totv * (1.0 / BN_CNT) + 1e-5)
            for p in range(P):
                oref[:, p * C:(p + 1) * C] = hn[p] * scale + bt

    return body


def kernel(x1, x2, adj1, adj2, Wx_w, Wx_b, Wn_w, Wn_b, Wr_w, Wr_b, gamma, beta):
    B, N, P, D = x1.shape
    DOUT = Wx_w.shape[0]
    C = 3 * DOUT

    x1p = x1.reshape(B * N, P * D)
    x2p = x2.reshape(B * N, P * D)

    out1, out2 = pl.pallas_call(
        _make_body(B, N, P, D, DOUT),
        out_shape=(
            jax.ShapeDtypeStruct((B * N, P * C), jnp.float32),
            jax.ShapeDtypeStruct((B * N, P * C), jnp.float32),
        ),
    )(x1p, x2p,
      Wx_w.T, Wn_w.T, Wr_w.T,
      Wx_b.reshape(1, DOUT), Wn_b.reshape(1, DOUT), Wr_b.reshape(1, DOUT),
      gamma.reshape(1, C), beta.reshape(1, C))

    return (out1.reshape(B, N, P, C), out2.reshape(B, N, P, C))


# speedup vs baseline: 4.2662x; 3.6071x over previous
import jax
import jax.numpy as jnp
from jax.experimental import pallas as pl

_F32 = jnp.float32


def _make_body(B, N, P, D, DOUT):
    BLK = N
    PB = P * N
    M = B * PB

    def body(x1_ref, x2_ref, wx_ref, wn_ref, wr_ref,
             bx_ref, bn_ref, br_ref, g_ref, bt_ref,
             o1_ref, o2_ref):
        att1 = []
        att2t = []
        for b in range(B):
            num = jnp.zeros((N, N), _F32)
            sq1 = jnp.zeros((N, 1), _F32)
            sq2 = jnp.zeros((N, 1), _F32)
            for p in range(P):
                r = b * PB + p * BLK
                a1 = x1_ref[r:r + BLK, :]
                a2 = x2_ref[r:r + BLK, :]
                num += jax.lax.dot_general(
                    a1, a2, (((1,), (1,)), ((), ())),
                    preferred_element_type=_F32)
                sq1 += jnp.sum(a1 * a1, axis=1, keepdims=True)
                sq2 += jnp.sum(a2 * a2, axis=1, keepdims=True)
            n1 = jnp.maximum(jnp.sqrt(sq1), 1e-6)
            n2 = jnp.maximum(jnp.sqrt(sq2), 1e-6)
            sim = num / (n1 * n2.T)
            m1 = jnp.max(sim, axis=1, keepdims=True)
            e1 = jnp.exp(sim - m1)
            att1.append(e1 / jnp.sum(e1, axis=1, keepdims=True))
            m2 = jnp.max(sim, axis=0, keepdims=True)
            e2 = jnp.exp(sim - m2)
            att2t.append(e2 / jnp.sum(e2, axis=0, keepdims=True))

        wx = wx_ref[:]
        wn = wn_ref[:]
        wr = wr_ref[:]
        bx = bx_ref[:]
        bn = bn_ref[:]
        br = br_ref[:]
        g = g_ref[:]
        bt = bt_ref[:]
        inv = 1.0 / (N - 1)

        for X, att, tr, oref in ((x1_ref[:], att1, False, o1_ref),
                                 (x2_ref[:], att2t, True, o2_ref)):
            S = jnp.dot(X, wx, preferred_element_type=_F32) + bx
            rn2 = jnp.sum(S * S, axis=1, keepdims=True)
            oref[:, 0:DOUT] = S

            Z = jnp.dot(X, wn, preferred_element_type=_F32)
            Z3 = Z.reshape(B * P, BLK, DOUT)
            s = jnp.sum(Z3, axis=1, keepdims=True)
            XN = ((s - Z3) * inv).reshape(M, DOUT) + bn
            rn2 += jnp.sum(XN * XN, axis=1, keepdims=True)
            oref[:, DOUT:2 * DOUT] = XN

            Zs = jnp.dot(X, wr, preferred_element_type=_F32)
            if not tr:
                Zo = jnp.dot(x2_ref[:], wr, preferred_element_type=_F32)
            else:
                Zo = jnp.dot(x1_ref[:], wr, preferred_element_type=_F32)
            mu_parts = []
            for b in range(B):
                a = att[b]
                for p in range(P):
                    r = b * PB + p * BLK
                    if not tr:
                        c = jnp.dot(a, Zo[r:r + BLK, :],
                                    preferred_element_type=_F32)
                    else:
                        c = jax.lax.dot_general(
                            a, Zo[r:r + BLK, :], (((0,), (0,)), ((), ())),
                            preferred_element_type=_F32)
                    mu_parts.append(Zs[r:r + BLK, :] - c + br)
            MUS = jnp.concatenate(mu_parts, axis=0)
            rn2 += jnp.sum(MUS * MUS, axis=1, keepdims=True)
            oref[:, 2 * DOUT:3 * DOUT] = MUS

            rinv = 1.0 / jnp.maximum(jnp.sqrt(rn2), 1e-12)
            h = jnp.maximum(oref[:] * rinv, 0.0)
            oref[:] = h
            sm = jnp.sum(h, axis=0, keepdims=True)
            sq = jnp.sum(h * h, axis=0, keepdims=True)
            mean = sm * (1.0 / M)
            var = jnp.maximum(sq * (1.0 / M) - mean * mean, 0.0)
            scale = g * jax.lax.rsqrt(var + 1e-5)
            shift = bt - mean * scale
            oref[:] = oref[:] * scale + shift

    return body


def kernel(x1, x2, adj1, adj2, Wx_w, Wx_b, Wn_w, Wn_b, Wr_w, Wr_b, gamma, beta):
    B, N, P, D = x1.shape
    DOUT = Wx_w.shape[0]
    C = 3 * DOUT
    M = B * N * P

    x1p = x1.transpose(0, 2, 1, 3).reshape(M, D)
    x2p = x2.transpose(0, 2, 1, 3).reshape(M, D)

    out1, out2 = pl.pallas_call(
        _make_body(B, N, P, D, DOUT),
        out_shape=(
            jax.ShapeDtypeStruct((M, C), jnp.float32),
            jax.ShapeDtypeStruct((M, C), jnp.float32),
        ),
    )(x1p, x2p,
      Wx_w.T, Wn_w.T, Wr_w.T,
      Wx_b.reshape(1, DOUT), Wn_b.reshape(1, DOUT), Wr_b.reshape(1, DOUT),
      gamma.reshape(1, C), beta.reshape(1, C))

    o1 = out1.reshape(B, P, N, C).transpose(0, 2, 1, 3)
    o2 = out2.reshape(B, P, N, C).transpose(0, 2, 1, 3)
    return (o1, o2)
